# baseline (speedup 1.0000x reference)
import functools

import jax
import jax.numpy as jnp
from jax import lax
from jax.experimental import pallas as pl
from jax.experimental.pallas import tpu as pltpu
from jax.experimental.pallas import tpu_sc as plsc

B = 1024
T = 200
C = 256
H = 2
VPAD = 104
NW = 32
B_PER = B // NW


def _table_body(emb_ref, w_ref, b_ref, out_ref):
    out_ref[...] = lax.dot_general(
        emb_ref[...], w_ref[...],
        dimension_numbers=(((1,), (0,)), ((), ())),
        preferred_element_type=jnp.float32,
    ) + b_ref[...]


def _fused_table(emb_pad, w_lin, b_row):
    return pl.pallas_call(
        _table_body,
        out_shape=jax.ShapeDtypeStruct((VPAD, C), jnp.float32),
    )(emb_pad, w_lin, b_row)


@functools.partial(
    pl.kernel,
    mesh=plsc.VectorSubcoreMesh(core_axis_name="c", subcore_axis_name="s"),
    out_type=jax.ShapeDtypeStruct((B, H, T, C), jnp.float32),
    compiler_params=pltpu.CompilerParams(needs_layout_passes=False),
    scratch_types=[
        pltpu.VMEM((T,), jnp.int32),
        pltpu.VMEM((T,), jnp.int32),
        pltpu.VMEM((2, T, C), jnp.float32),
        pltpu.VMEM((VPAD * C,), jnp.float32),
        pltpu.SemaphoreType.DMA((2,)),
        pltpu.SemaphoreType.DMA((2,)),
    ],
)
def _sc_lookup(tv_hbm, x_hbm, out_hbm, xa_v, xb_v, stg_v, tv_v, xsem, wsem):
    wid = lax.axis_index("s") * 2 + lax.axis_index("c")
    b0 = wid * B_PER
    civ = lax.iota(jnp.int32, 16)
    pltpu.sync_copy(tv_hbm, tv_v)
    xbufs = (xa_v, xb_v)
    pltpu.async_copy(x_hbm.at[pl.ds(b0 * T, T)], xa_v, xsem.at[0])

    def xload(i, p):
        nxt = jnp.minimum(i + 1, B_PER - 1)
        pltpu.async_copy(
            x_hbm.at[pl.ds((b0 + nxt) * T, T)], xbufs[1 - p], xsem.at[1 - p])

    def pair_body(i2, carry):
        for p in range(2):
            i = i2 * 2 + p
            b = b0 + i
            stg = stg_v.at[p]
            writes = [
                pltpu.make_async_copy(stg, out_hbm.at[b, h], wsem.at[p])
                for h in range(H)
            ]

            pltpu.make_async_copy(
                x_hbm.at[pl.ds(b * T, T)], xbufs[p], xsem.at[p]).wait()
            xload(i, p)

            @pl.when(i2 > 0)
            def _drain():
                for w in writes:
                    w.wait()

            @plsc.parallel_loop(0, T, 1, unroll=4)
            def _fill(w):
                xsp = plsc.load_gather(xbufs[p], [jnp.full((16,), w, jnp.int32)])
                base = xsp * C
                for k in range(C // 16):
                    lanes = base + (civ + (k * 16))
                    stg_v[p, w, pl.ds(k * 16, 16)] = plsc.load_gather(
                        tv_v, [lanes])

            for w in writes:
                w.start()
        return carry

    lax.fori_loop(0, B_PER // 2, pair_body, 0)

    pltpu.make_async_copy(
        x_hbm.at[pl.ds((b0 + B_PER - 1) * T, T)], xa_v, xsem.at[0]).wait()
    last = b0 + B_PER - 2
    for p in range(2):
        for h in range(H):
            pltpu.make_async_copy(
                stg_v.at[p], out_hbm.at[last + p, h], wsem.at[p]).wait()


def kernel(x, f_xs_shape, embed_table, W_lin, b_lin):
    emb_pad = jnp.zeros((VPAD, embed_table.shape[1]), jnp.float32)
    emb_pad = emb_pad.at[: embed_table.shape[0]].set(embed_table)
    tv = _fused_table(emb_pad, W_lin, b_lin.reshape(1, C))
    out = _sc_lookup(tv.reshape(-1), x.reshape(-1))
    return jnp.transpose(out, (0, 3, 1, 2))
